# baseline (speedup 1.0000x reference)
import jax
import jax.numpy as jnp
from jax.experimental import pallas as pl
from jax.experimental.pallas import tpu as pltpu

B, N, H, E = 8, 1024, 128, 8
NT = 4
TN = N // NT
EPS = 1e-5


def _moe_gcn_kernel(x_ref, adj_ref, rw_ref, rb_ref, ws_ref, bnw_ref, bnb_ref,
                    out_ref, onehot_scr, s1_scr, s2_scr, sup_scr):
    b = pl.program_id(0)
    t = pl.program_id(1)

    @pl.when(t == 0)
    def _route_and_support():
        xm = jnp.mean(x_ref[0], axis=0, keepdims=True)
        scores = jax.lax.dot_general(
            xm, rw_ref[...], (((1,), (1,)), ((), ())),
            preferred_element_type=jnp.float32) + rb_ref[...]
        iota = jax.lax.broadcasted_iota(jnp.int32, (1, E), 1)
        mx = jnp.max(scores, axis=1, keepdims=True)
        first = jnp.min(jnp.where(scores == mx, iota, E), axis=1,
                        keepdims=True)
        oh = (iota == first).astype(jnp.float32)
        onehot_scr[b] = oh[0]
        w = jnp.sum(ws_ref[...] * oh[0][:, None, None], axis=0)
        sup_scr[...] = jnp.dot(
            x_ref[0], w, preferred_element_type=jnp.float32
        ).astype(jnp.bfloat16)

    o = jnp.dot(adj_ref[0].astype(jnp.bfloat16), sup_scr[...],
                preferred_element_type=jnp.float32)
    out_ref[b, pl.ds(t * TN, TN), :] = o

    part1 = jnp.sum(o, axis=0)
    part2 = jnp.sum(o * o, axis=0)

    @pl.when(t == 0)
    def _init_partials():
        s1_scr[b] = part1
        s2_scr[b] = part2

    @pl.when(t != 0)
    def _acc_partials():
        s1_scr[b] += part1
        s2_scr[b] += part2

    @pl.when((b == B - 1) & (t == NT - 1))
    def _bn_epilogue():
        oh_all = onehot_scr[...]
        same = jax.lax.dot_general(
            oh_all, oh_all, (((1,), (1,)), ((), ())),
            preferred_element_type=jnp.float32)
        cnt = jnp.maximum(jnp.sum(same, axis=1, keepdims=True) * N, 1.0)
        g1 = jnp.dot(same, s1_scr[...], preferred_element_type=jnp.float32)
        g2 = jnp.dot(same, s2_scr[...], preferred_element_type=jnp.float32)
        mean = g1 / cnt
        var = jnp.maximum(g2 / cnt - mean * mean, 0.0)
        gamma = jnp.dot(oh_all, bnw_ref[...], preferred_element_type=jnp.float32)
        beta = jnp.dot(oh_all, bnb_ref[...], preferred_element_type=jnp.float32)
        scale = gamma * jax.lax.rsqrt(var + EPS)
        shift = beta - mean * scale
        out_ref[...] = jnp.maximum(
            out_ref[...] * scale[:, None, :] + shift[:, None, :], 0.0)


@jax.jit
def kernel(x, adj, router_w, router_b, Ws, bn_w, bn_b):
    grid_spec = pltpu.PrefetchScalarGridSpec(
        num_scalar_prefetch=0,
        grid=(B, NT),
        in_specs=[
            pl.BlockSpec((1, N, H), lambda b, t: (b, 0, 0)),
            pl.BlockSpec((1, TN, N), lambda b, t: (b, t, 0)),
            pl.BlockSpec((E, H), lambda b, t: (0, 0)),
            pl.BlockSpec((1, E), lambda b, t: (0, 0)),
            pl.BlockSpec((E, H, H), lambda b, t: (0, 0, 0)),
            pl.BlockSpec((E, H), lambda b, t: (0, 0)),
            pl.BlockSpec((E, H), lambda b, t: (0, 0)),
        ],
        out_specs=pl.BlockSpec((B, N, H), lambda b, t: (0, 0, 0)),
        scratch_shapes=[
            pltpu.VMEM((B, E), jnp.float32),
            pltpu.VMEM((B, H), jnp.float32),
            pltpu.VMEM((B, H), jnp.float32),
            pltpu.VMEM((N, H), jnp.bfloat16),
        ],
    )
    return pl.pallas_call(
        _moe_gcn_kernel,
        grid_spec=grid_spec,
        out_shape=jax.ShapeDtypeStruct((B, N, H), jnp.float32),
        compiler_params=pltpu.CompilerParams(
            dimension_semantics=("arbitrary", "arbitrary"),
        ),
    )(x, adj, router_w, router_b.reshape(1, E), Ws, bn_w, bn_b)


# speedup vs baseline: 1.8639x; 1.8639x over previous
import jax
import jax.numpy as jnp
from jax.experimental import pallas as pl
from jax.experimental.pallas import tpu as pltpu

B, N, H, E = 8, 1024, 128, 8
EPS = 1e-5


def _moe_gcn_kernel(x_ref, adj_ref, rw_ref, rb_ref, ws_ref, bnw_ref, bnb_ref,
                    out_ref, onehot_scr, s1_scr, s2_scr):
    b = pl.program_id(0)

    xm = jnp.mean(x_ref[0], axis=0, keepdims=True)
    scores = jax.lax.dot_general(
        xm, rw_ref[...], (((1,), (1,)), ((), ())),
        preferred_element_type=jnp.float32) + rb_ref[...]
    iota = jax.lax.broadcasted_iota(jnp.int32, (1, E), 1)
    mx = jnp.max(scores, axis=1, keepdims=True)
    first = jnp.min(jnp.where(scores == mx, iota, E), axis=1, keepdims=True)
    oh = (iota == first).astype(jnp.float32)
    onehot_scr[b] = oh[0]

    w = jnp.sum(ws_ref[...] * oh[0][:, None, None], axis=0)

    support = jnp.dot(x_ref[0], w, preferred_element_type=jnp.float32)
    o = jnp.dot(adj_ref[0].astype(jnp.bfloat16),
                support.astype(jnp.bfloat16),
                preferred_element_type=jnp.float32)

    out_ref[b] = o
    s1_scr[b] = jnp.sum(o, axis=0)
    s2_scr[b] = jnp.sum(o * o, axis=0)

    @pl.when(b == B - 1)
    def _bn_epilogue():
        oh_all = onehot_scr[...]
        same = jax.lax.dot_general(
            oh_all, oh_all, (((1,), (1,)), ((), ())),
            preferred_element_type=jnp.float32)
        cnt = jnp.maximum(jnp.sum(same, axis=1, keepdims=True) * N, 1.0)
        g1 = jnp.dot(same, s1_scr[...], preferred_element_type=jnp.float32)
        g2 = jnp.dot(same, s2_scr[...], preferred_element_type=jnp.float32)
        mean = g1 / cnt
        var = jnp.maximum(g2 / cnt - mean * mean, 0.0)
        gamma = jnp.dot(oh_all, bnw_ref[...], preferred_element_type=jnp.float32)
        beta = jnp.dot(oh_all, bnb_ref[...], preferred_element_type=jnp.float32)
        scale = gamma * jax.lax.rsqrt(var + EPS)
        shift = beta - mean * scale
        out_ref[...] = jnp.maximum(
            out_ref[...] * scale[:, None, :] + shift[:, None, :], 0.0)


@jax.jit
def kernel(x, adj, router_w, router_b, Ws, bn_w, bn_b):
    grid_spec = pltpu.PrefetchScalarGridSpec(
        num_scalar_prefetch=0,
        grid=(B,),
        in_specs=[
            pl.BlockSpec((1, N, H), lambda b: (b, 0, 0)),
            pl.BlockSpec((1, N, N), lambda b: (b, 0, 0)),
            pl.BlockSpec((E, H), lambda b: (0, 0)),
            pl.BlockSpec((1, E), lambda b: (0, 0)),
            pl.BlockSpec((E, H, H), lambda b: (0, 0, 0)),
            pl.BlockSpec((E, H), lambda b: (0, 0)),
            pl.BlockSpec((E, H), lambda b: (0, 0)),
        ],
        out_specs=pl.BlockSpec((B, N, H), lambda b: (0, 0, 0)),
        scratch_shapes=[
            pltpu.VMEM((B, E), jnp.float32),
            pltpu.VMEM((B, H), jnp.float32),
            pltpu.VMEM((B, H), jnp.float32),
        ],
    )
    return pl.pallas_call(
        _moe_gcn_kernel,
        grid_spec=grid_spec,
        out_shape=jax.ShapeDtypeStruct((B, N, H), jnp.float32),
        compiler_params=pltpu.CompilerParams(
            dimension_semantics=("arbitrary",),
        ),
    )(x, adj, router_w, router_b.reshape(1, E), Ws, bn_w, bn_b)
